# Initial kernel scaffold; baseline (speedup 1.0000x reference)
import jax
import jax.numpy as jnp
from jax.experimental import pallas as pl


def kernel(index, output, target, pred_hist):
    raise NotImplementedError("write your pallas kernel here")


# speedup vs baseline: 7.5120x; 7.5120x over previous
import jax
import jax.numpy as jnp
from jax.experimental import pallas as pl
from jax.experimental.pallas import tpu as pltpu

_W0 = 1.0 / 1223
_W1 = 1.0 / 2444
_W2 = 1.0 / 1687


def _ce_kernel(x_ref, t_ref, loss_ref):
    x = x_ref[...]
    t = t_ref[...]
    x0, x1, x2 = x[0], x[1], x[2]
    m = jnp.maximum(jnp.maximum(x0, x1), x2)
    e0 = jnp.exp(x0 - m)
    e1 = jnp.exp(x1 - m)
    e2 = jnp.exp(x2 - m)
    lse = m + jnp.log(e0 + e1 + e2)
    is0 = t == 0
    is1 = t == 1
    picked = jnp.where(is0, x0, jnp.where(is1, x1, x2)) - lse
    w = jnp.where(is0, _W0, jnp.where(is1, _W1, _W2)).astype(jnp.float32)
    num = jnp.sum(w * picked)
    den = jnp.sum(w)
    loss_ref[0, 0] = -(num / den)


def kernel(index, output, target, pred_hist):
    del index, pred_hist
    x = output.T.reshape(3, 128, 128)
    t = target.reshape(128, 128)
    loss = pl.pallas_call(
        _ce_kernel,
        out_shape=jax.ShapeDtypeStruct((1, 1), jnp.float32),
        out_specs=pl.BlockSpec(memory_space=pltpu.SMEM),
    )(x, t)
    return loss[0, 0]
